# baseline (speedup 1.0000x reference)
import functools

import jax
import jax.numpy as jnp
from jax import lax
from jax.experimental import pallas as pl
from jax.experimental.pallas import tpu as pltpu
from jax.experimental.pallas import tpu_sc as plsc

EMB_DIM = 32
BATCH = 16384
NUSERS = 1000001
NJOKES = 100001
NC = 2
NS = 16
NW = NC * NS
B_PER_W = BATCH // NW
L = 16
NGRP = B_PER_W // L
GCH = 128
NIDX = B_PER_W * EMB_DIM


def _rsqrt16(x):
    i = lax.bitcast_convert_type(x, jnp.int32)
    y = lax.bitcast_convert_type(jnp.int32(0x5F3759DF) - (i >> 1), jnp.float32)
    for _ in range(3):
        y = y * (jnp.float32(1.5) - jnp.float32(0.5) * x * y * y)
    return y


def _make_kernel():
    mesh = plsc.VectorSubcoreMesh(core_axis_name="c", subcore_axis_name="s")

    @functools.partial(
        pl.kernel,
        mesh=mesh,
        compiler_params=pltpu.CompilerParams(
            needs_layout_passes=False, use_tc_tiling_on_sc=False),
        out_type=jax.ShapeDtypeStruct((BATCH,), jnp.float32),
        scratch_types=[
            pltpu.VMEM((B_PER_W,), jnp.int32),
            pltpu.VMEM((B_PER_W,), jnp.int32),
            pltpu.VMEM((NIDX,), jnp.int32),
            pltpu.VMEM((NIDX,), jnp.int32),
            pltpu.VMEM((NIDX,), jnp.float32),
            pltpu.VMEM((NIDX,), jnp.float32),
            pltpu.VMEM((B_PER_W,), jnp.float32),
            pltpu.SemaphoreType.DMA,
            pltpu.SemaphoreType.DMA,
            pltpu.SemaphoreType.DMA,
        ],
    )
    def cosine_kernel(uids_hbm, jids_hbm, utab_hbm, jtab_hbm, out_hbm,
                      uidx_v, jidx_v, ufi_v, jfi_v, uel_v, jel_v, outv,
                      sem_i, sem_u, sem_j):
        wid = lax.axis_index("s") * NC + lax.axis_index("c")
        base = wid * B_PER_W

        ci_u = pltpu.async_copy(uids_hbm.at[pl.ds(base, B_PER_W)], uidx_v, sem_i)
        ci_j = pltpu.async_copy(jids_hbm.at[pl.ds(base, B_PER_W)], jidx_v, sem_i)
        ci_u.wait()
        ci_j.wait()

        def idx_body(g, carry):
            uvec = uidx_v[pl.ds(g * L, L)]
            jvec = jidx_v[pl.ds(g * L, L)]
            for f in range(EMB_DIM):
                ufi_v[pl.ds(f * B_PER_W + g * L, L)] = uvec + f * NUSERS
                jfi_v[pl.ds(f * B_PER_W + g * L, L)] = jvec + f * NJOKES
            return carry

        lax.fori_loop(0, NGRP, idx_body, 0)

        ucopies = []
        jcopies = []
        for c in range(NIDX // GCH):
            sl = pl.ds(c * GCH, GCH)
            ucopies.append(pltpu.async_copy(
                utab_hbm.at[ufi_v.at[sl]], uel_v.at[sl], sem_u))
            jcopies.append(pltpu.async_copy(
                jtab_hbm.at[jfi_v.at[sl]], jel_v.at[sl], sem_j))
        for cp in ucopies:
            cp.wait()
        for cp in jcopies:
            cp.wait()

        zeros = jnp.zeros((L,), jnp.float32)
        eps = jnp.float32(1e-12)

        def group_body(g, carry):
            d = zeros
            uu = zeros
            jj = zeros
            for f in range(EMB_DIM):
                uf = uel_v[pl.ds(f * B_PER_W + g * L, L)]
                jf = jel_v[pl.ds(f * B_PER_W + g * L, L)]
                d = d + uf * jf
                uu = uu + uf * uf
                jj = jj + jf * jf
            uu = jnp.maximum(uu, eps)
            jj = jnp.maximum(jj, eps)
            outv[pl.ds(g * L, L)] = d * _rsqrt16(uu) * _rsqrt16(jj)
            return carry

        lax.fori_loop(0, NGRP, group_body, 0)
        pltpu.sync_copy(outv, out_hbm.at[pl.ds(base, B_PER_W)])

    return cosine_kernel


_kernel_call = _make_kernel()


def kernel(user_ids, joke_ids, user_table, joke_table):
    ut_flat = user_table.T.reshape(-1)
    jt_flat = joke_table.T.reshape(-1)
    out = _kernel_call(user_ids, joke_ids, ut_flat, jt_flat)
    return out.reshape(BATCH, 1)
